# Initial kernel scaffold; baseline (speedup 1.0000x reference)
import jax
import jax.numpy as jnp
from jax.experimental import pallas as pl


def kernel(seq, att, segment_ids):
    raise NotImplementedError("write your pallas kernel here")


# speedup vs baseline: 5.4994x; 5.4994x over previous
import functools

import jax
import jax.numpy as jnp
from jax.experimental import pallas as pl
from jax.experimental.pallas import tpu as pltpu

NUM_SEGMENTS = 16
TOTAL_TOKENS = 32768
DIM = 1024
BLOCK_T = 2048
NEG = -1e30


def _body(x_ref, att_ref, idc_ref, idr_ref, out_ref, m_ref, d_ref, acc_ref):
    i = pl.program_id(0)
    nb = pl.num_programs(0)
    S = NUM_SEGMENTS
    T = BLOCK_T

    @pl.when(i == 0)
    def _init():
        m_ref[...] = jnp.full((1, S), NEG, jnp.float32)
        d_ref[...] = jnp.zeros((1, S), jnp.float32)
        acc_ref[...] = jnp.zeros((S, DIM), jnp.float32)

    x = x_ref[...]
    a = att_ref[...]
    idc = idc_ref[...]
    idr = idr_ref[0]

    l = jnp.dot(x, a, preferred_element_type=jnp.float32)

    seg_ts = jax.lax.broadcasted_iota(jnp.int32, (T, S), 1)
    mask_ts = seg_ts == idc

    bm = jnp.max(jnp.where(mask_ts, l, NEG), axis=0, keepdims=True)
    m_old = m_ref[...]
    d_old = d_ref[...]
    m_new = jnp.maximum(m_old, bm)
    c = jnp.exp(m_old - m_new)

    m_tok = jnp.sum(jnp.where(mask_ts, m_new, 0.0), axis=1, keepdims=True)
    e = jnp.exp(l - m_tok)

    d_inc = jnp.sum(jnp.where(mask_ts, e, 0.0), axis=0, keepdims=True)
    m_ref[...] = m_new
    d_ref[...] = d_old * c + d_inc

    seg_st = jax.lax.broadcasted_iota(jnp.int32, (S, T), 0)
    p = jnp.where(seg_st == idr, 1.0, 0.0)
    eye = (jax.lax.broadcasted_iota(jnp.int32, (S, S), 0)
           == jax.lax.broadcasted_iota(jnp.int32, (S, S), 1))
    cd = jnp.where(eye, c, 0.0)
    acc_ref[...] = (jnp.dot(cd, acc_ref[...], preferred_element_type=jnp.float32)
                    + jnp.dot(p, e * x, preferred_element_type=jnp.float32))

    @pl.when(i == nb - 1)
    def _fin():
        d = d_ref[...]
        dinv = jnp.where(eye, jnp.where(d > 0, 1.0 / d, 0.0), 0.0)
        out_ref[...] = jnp.dot(dinv, acc_ref[...],
                               preferred_element_type=jnp.float32)


@jax.jit
def kernel(seq, att, segment_ids):
    ids = segment_ids.astype(jnp.int32)
    nb = TOTAL_TOKENS // BLOCK_T
    idc = ids.reshape(TOTAL_TOKENS, 1)
    idr = ids.reshape(nb, 1, BLOCK_T)
    return pl.pallas_call(
        _body,
        grid=(nb,),
        in_specs=[
            pl.BlockSpec((BLOCK_T, DIM), lambda i: (i, 0)),
            pl.BlockSpec((DIM, 1), lambda i: (0, 0)),
            pl.BlockSpec((BLOCK_T, 1), lambda i: (i, 0)),
            pl.BlockSpec((1, 1, BLOCK_T), lambda i: (i, 0, 0)),
        ],
        out_specs=pl.BlockSpec((NUM_SEGMENTS, DIM), lambda i: (0, 0)),
        out_shape=jax.ShapeDtypeStruct((NUM_SEGMENTS, DIM), jnp.float32),
        scratch_shapes=[
            pltpu.VMEM((1, NUM_SEGMENTS), jnp.float32),
            pltpu.VMEM((1, NUM_SEGMENTS), jnp.float32),
            pltpu.VMEM((NUM_SEGMENTS, DIM), jnp.float32),
        ],
        compiler_params=pltpu.CompilerParams(
            dimension_semantics=("arbitrary",)),
    )(seq, att, idc, idr)


# speedup vs baseline: 8.5498x; 1.5547x over previous
import functools

import jax
import jax.numpy as jnp
from jax.experimental import pallas as pl
from jax.experimental.pallas import tpu as pltpu

NUM_SEGMENTS = 16
TOTAL_TOKENS = 32768
DIM = 1024
BLOCK_T = 2048
NEG = -1e30


def _body(x_ref, att_ref, idr_ref, out_ref, m_ref, d_ref, acc_ref):
    i = pl.program_id(0)
    nb = pl.num_programs(0)
    S = NUM_SEGMENTS
    T = BLOCK_T

    @pl.when(i == 0)
    def _init():
        m_ref[...] = jnp.full((S, 1), NEG, jnp.float32)
        d_ref[...] = jnp.zeros((S, 1), jnp.float32)
        acc_ref[...] = jnp.zeros((S, DIM), jnp.float32)

    x = x_ref[...]
    a = att_ref[...]
    idr = idr_ref[0]

    l = jax.lax.dot_general(a, x, (((1,), (1,)), ((), ())),
                            preferred_element_type=jnp.float32)

    seg_st = jax.lax.broadcasted_iota(jnp.int32, (S, T), 0)
    mask = seg_st == idr
    lm = jnp.where(mask, l, NEG)
    bm = jnp.max(lm, axis=1, keepdims=True)
    m_old = m_ref[...]
    m_new = jnp.maximum(m_old, bm)
    c = jnp.exp(m_old - m_new)
    pw = jnp.exp(jnp.where(mask, l - m_new, NEG))
    d_ref[...] = d_ref[...] * c + jnp.sum(pw, axis=1, keepdims=True)
    m_ref[...] = m_new
    acc_ref[...] = (acc_ref[...] * c
                    + jnp.dot(pw, x, preferred_element_type=jnp.float32))

    @pl.when(i == nb - 1)
    def _fin():
        d = d_ref[...]
        out_ref[...] = jnp.where(d > 0, acc_ref[...] / d, 0.0)


@jax.jit
def kernel(seq, att, segment_ids):
    ids = segment_ids.astype(jnp.int32)
    nb = TOTAL_TOKENS // BLOCK_T
    idr = ids.reshape(nb, 1, BLOCK_T)
    att_row = att.reshape(1, DIM)
    return pl.pallas_call(
        _body,
        grid=(nb,),
        in_specs=[
            pl.BlockSpec((BLOCK_T, DIM), lambda i: (i, 0)),
            pl.BlockSpec((1, DIM), lambda i: (0, 0)),
            pl.BlockSpec((1, 1, BLOCK_T), lambda i: (i, 0, 0)),
        ],
        out_specs=pl.BlockSpec((NUM_SEGMENTS, DIM), lambda i: (0, 0)),
        out_shape=jax.ShapeDtypeStruct((NUM_SEGMENTS, DIM), jnp.float32),
        scratch_shapes=[
            pltpu.VMEM((NUM_SEGMENTS, 1), jnp.float32),
            pltpu.VMEM((NUM_SEGMENTS, 1), jnp.float32),
            pltpu.VMEM((NUM_SEGMENTS, DIM), jnp.float32),
        ],
        compiler_params=pltpu.CompilerParams(
            dimension_semantics=("arbitrary",)),
    )(seq, att_row, idr)


# speedup vs baseline: 8.7749x; 1.0263x over previous
import functools

import jax
import jax.numpy as jnp
from jax.experimental import pallas as pl
from jax.experimental.pallas import tpu as pltpu

NUM_SEGMENTS = 16
TOTAL_TOKENS = 32768
DIM = 1024
BLOCK_T = 4096
NEG = -1e30


def _body(x_ref, att_ref, idr_ref, out_ref, m_ref, d_ref, acc_ref):
    i = pl.program_id(0)
    nb = pl.num_programs(0)
    S = NUM_SEGMENTS
    T = BLOCK_T

    @pl.when(i == 0)
    def _init():
        m_ref[...] = jnp.full((S, 1), NEG, jnp.float32)
        d_ref[...] = jnp.zeros((S, 1), jnp.float32)
        acc_ref[...] = jnp.zeros((S, DIM), jnp.float32)

    x = x_ref[...]
    a = att_ref[...]
    idr = idr_ref[0]

    l = jax.lax.dot_general(a, x, (((1,), (1,)), ((), ())),
                            preferred_element_type=jnp.float32)

    seg_st = jax.lax.broadcasted_iota(jnp.int32, (S, T), 0)
    mask = seg_st == idr
    lm = jnp.where(mask, l, NEG)
    bm = jnp.max(lm, axis=1, keepdims=True)
    m_old = m_ref[...]
    m_new = jnp.maximum(m_old, bm)
    c = jnp.exp(m_old - m_new)
    pw = jnp.exp(jnp.where(mask, l - m_new, NEG))
    d_ref[...] = d_ref[...] * c + jnp.sum(pw, axis=1, keepdims=True)
    m_ref[...] = m_new
    acc_ref[...] = (acc_ref[...] * c
                    + jnp.dot(pw, x, preferred_element_type=jnp.float32))

    @pl.when(i == nb - 1)
    def _fin():
        d = d_ref[...]
        out_ref[...] = jnp.where(d > 0, acc_ref[...] / d, 0.0)


@jax.jit
def kernel(seq, att, segment_ids):
    ids = segment_ids.astype(jnp.int32)
    nb = TOTAL_TOKENS // BLOCK_T
    idr = ids.reshape(nb, 1, BLOCK_T)
    att_row = att.reshape(1, DIM)
    return pl.pallas_call(
        _body,
        grid=(nb,),
        in_specs=[
            pl.BlockSpec((BLOCK_T, DIM), lambda i: (i, 0)),
            pl.BlockSpec((1, DIM), lambda i: (0, 0)),
            pl.BlockSpec((1, 1, BLOCK_T), lambda i: (i, 0, 0)),
        ],
        out_specs=pl.BlockSpec((NUM_SEGMENTS, DIM), lambda i: (0, 0)),
        out_shape=jax.ShapeDtypeStruct((NUM_SEGMENTS, DIM), jnp.float32),
        scratch_shapes=[
            pltpu.VMEM((NUM_SEGMENTS, 1), jnp.float32),
            pltpu.VMEM((NUM_SEGMENTS, 1), jnp.float32),
            pltpu.VMEM((NUM_SEGMENTS, DIM), jnp.float32),
        ],
        compiler_params=pltpu.CompilerParams(
            dimension_semantics=("arbitrary",)),
    )(seq, att_row, idr)
